# baseline (speedup 1.0000x reference)
import functools

import jax
import jax.numpy as jnp
from jax import lax
from jax.experimental import pallas as pl
from jax.experimental.pallas import tpu as pltpu
from jax.experimental.pallas import tpu_sc as plsc

VOCAB = 8192
D = 8192
NC, NS = 2, 16
NW = NC * NS
BTOT = 16 * 256
BPW = BTOT // NW
K = 4
NCH = BPW // K


def _gather_body(idx_hbm, tbl_hbm, out_hbm, idx_v, buf0, buf1,
                 sg0, sg1, so0, so1):
    wid = lax.axis_index("s") * NC + lax.axis_index("c")
    base = wid * BPW
    pltpu.sync_copy(idx_hbm.at[wid], idx_v)
    bufs = (buf0, buf1)
    sgs = (sg0, sg1)
    sos = (so0, so1)

    def start_g(c, b):
        pltpu.async_copy(tbl_hbm.at[idx_v.at[c]], bufs[b], sgs[b])

    def wait_g(c, b):
        pltpu.make_async_copy(tbl_hbm.at[idx_v.at[c]], bufs[b], sgs[b]).wait()

    def start_o(c, b):
        pltpu.async_copy(bufs[b], out_hbm.at[pl.ds(base + c * K, K)], sos[b])

    def wait_o(c, b):
        pltpu.make_async_copy(
            bufs[b], out_hbm.at[pl.ds(base + c * K, K)], sos[b]).wait()

    start_g(0, 0)
    start_g(1, 1)
    wait_g(0, 0)
    start_o(0, 0)

    def pair_body(i, carry):
        c = 2 * i + 1
        wait_o(c - 1, 0)
        start_g(c + 1, 0)
        wait_g(c, 1)
        start_o(c, 1)
        c2 = c + 1
        wait_o(c2 - 1, 1)
        start_g(c2 + 1, 1)
        wait_g(c2, 0)
        start_o(c2, 0)
        return carry

    lax.fori_loop(0, (NCH - 2) // 2, pair_body, 0)

    wait_o(NCH - 2, 0)
    wait_g(NCH - 1, 1)
    start_o(NCH - 1, 1)
    wait_o(NCH - 1, 1)


_sc_gather = functools.partial(
    pl.kernel,
    mesh=plsc.VectorSubcoreMesh(core_axis_name="c", subcore_axis_name="s"),
    out_type=jax.ShapeDtypeStruct((BTOT, D), jnp.float32),
    scratch_types=[
        pltpu.VMEM((NCH, K), jnp.int32),
        pltpu.VMEM((K, D), jnp.float32),
        pltpu.VMEM((K, D), jnp.float32),
        pltpu.SemaphoreType.DMA,
        pltpu.SemaphoreType.DMA,
        pltpu.SemaphoreType.DMA,
        pltpu.SemaphoreType.DMA,
    ],
)(_gather_body)


def kernel(x, embed_weight):
    B, L = x.shape
    idx = x.reshape(NW, NCH, K).astype(jnp.int32)
    out = _sc_gather(idx, embed_weight)
    return out.reshape(B, L, D)
